# baseline (speedup 1.0000x reference)
import functools

import jax
import jax.numpy as jnp
from jax import lax
from jax.experimental import pallas as pl
from jax.experimental.pallas import tpu as pltpu
from jax.experimental.pallas import tpu_sc as plsc

N_NODES = 49152
F = 64
NVARS = 4
B = 2
V = 4

_NC = 2
_NS = 16
_NW = _NC * _NS

_NH = N_NODES // 2
_RH = NVARS * _NH
_NPW = _RH // _NW
_IDX_PER_XFER = 128
_XFERS_PER_BLK = 8
_BLK_ROWS = _IDX_PER_XFER * _XFERS_PER_BLK
_NBLK = _NPW // _BLK_ROWS
_XPW = _NPW // _IDX_PER_XFER


def _sc_gather_paired_half(emb, idx3):
    mesh = plsc.VectorSubcoreMesh(core_axis_name="c", subcore_axis_name="s")

    @functools.partial(
        pl.kernel,
        out_type=jax.ShapeDtypeStruct((NVARS // 2, _NH, 2 * F), jnp.float32),
        mesh=mesh,
        compiler_params=pltpu.CompilerParams(use_tc_tiling_on_sc=False),
        scratch_types=[
            pltpu.VMEM((_XPW, _IDX_PER_XFER), jnp.int32),
            pltpu.VMEM((_BLK_ROWS, F), jnp.float32),
            pltpu.SemaphoreType.DMA,
        ],
    )
    def gather_kernel(emb_hbm, idx_hbm, out_hbm, idx_v, rows_v, sem):
        wid = lax.axis_index("s") * _NC + lax.axis_index("c")
        u = wid // 8
        node0 = (wid % 8) * _NPW
        p = u // 2
        q = u % 2
        pltpu.sync_copy(idx_hbm.at[wid], idx_v)

        def blk_body(blk, _):
            copies = []
            for j in range(_XFERS_PER_BLK):
                copies.append(pltpu.async_copy(
                    emb_hbm.at[u].at[idx_v.at[blk * _XFERS_PER_BLK + j]],
                    rows_v.at[pl.ds(j * _IDX_PER_XFER, _IDX_PER_XFER)],
                    sem,
                ))
            for c in copies:
                c.wait()
            pltpu.sync_copy(
                rows_v,
                out_hbm.at[p, pl.ds(node0 + blk * _BLK_ROWS, _BLK_ROWS),
                           pl.ds(q * F, F)],
            )
            return ()

        lax.fori_loop(0, _NBLK, blk_body, (), unroll=False)

    return gather_kernel(emb, idx3)


_BNT = 4096


def _detile_body(e_ref, o_ref):
    o_ref[0] = jnp.swapaxes(e_ref[0], 0, 1)


def _tc_detile(embT):
    return pl.pallas_call(
        _detile_body,
        grid=(NVARS, N_NODES // _BNT),
        in_specs=[pl.BlockSpec((1, F, _BNT), lambda u, n: (u, 0, n))],
        out_specs=pl.BlockSpec((1, _BNT, F), lambda u, n: (u, n, 0)),
        out_shape=jax.ShapeDtypeStruct((NVARS, N_NODES, F), jnp.float32),
    )(embT)


_BN = 8192
_NBH = _NH // _BN


def _film_body(vi_ref, x_ref, eg_ref, w_ref, b_ref, o_ref):
    bv = pl.program_id(1)
    vi = vi_ref[bv]
    pair = eg_ref[vi // 2]
    m_all = lax.dot_general(
        w_ref[...], pair, (((1,), (1,)), ((), ())),
        preferred_element_type=jnp.float32,
    )
    m = jnp.where(vi % 2 == 0, m_all[: 2 * F], m_all[2 * F :])
    m = m + b_ref[:, 0:1]
    o_ref[0, 0, 0] = x_ref[0, 0, 0] * m[:F] + m[F:]


def _film_body_alias(vi_ref, x_ref, eg_ref, w_ref, b_ref, prev_ref, o_ref):
    del prev_ref
    _film_body(vi_ref, x_ref, eg_ref, w_ref, b_ref, o_ref)


def _tc_film_half(vi, xT, eg, W2T, b2, half, prev=None):
    grid = (_NBH, B * V)
    base = half * _NBH

    in_specs = [
        pl.BlockSpec(
            (1, 1, 1, F, _BN),
            lambda n, bv, vi_ref: (bv // V, 0, bv % V, 0, base + n),
        ),
        pl.BlockSpec(
            (NVARS // 2, _BN, 2 * F),
            lambda n, bv, vi_ref: (0, n, 0),
        ),
        pl.BlockSpec((4 * F, 2 * F), lambda n, bv, vi_ref: (0, 0)),
        pl.BlockSpec((2 * F, 2 * F), lambda n, bv, vi_ref: (0, 0)),
    ]
    out_spec = pl.BlockSpec(
        (1, 1, 1, F, _BN),
        lambda n, bv, vi_ref: (bv // V, 0, bv % V, 0, base + n),
    )
    args = [vi, xT, eg, W2T, b2]
    kwargs = {}
    body = _film_body
    if prev is not None:
        in_specs.append(pl.BlockSpec(memory_space=pl.ANY))
        args.append(prev)
        kwargs["input_output_aliases"] = {5: 0}
        body = _film_body_alias
    grid_spec = pltpu.PrefetchScalarGridSpec(
        num_scalar_prefetch=1,
        grid=grid,
        in_specs=in_specs,
        out_specs=out_spec,
    )
    return pl.pallas_call(
        body,
        grid_spec=grid_spec,
        out_shape=jax.ShapeDtypeStruct(xT.shape, xT.dtype),
        **kwargs,
    )(*args)


def kernel(x, var_idx, adjc, embeddings, W, b):
    node_idx = adjc[:, 0].astype(jnp.int32)
    ones4 = jnp.ones((NVARS, 1), dtype=jnp.int32)
    idx0 = (ones4 * node_idx[None, :_NH]).reshape(_NW, _XPW, _IDX_PER_XFER)
    idx1 = (ones4 * node_idx[None, _NH:]).reshape(_NW, _XPW, _IDX_PER_XFER)
    table3 = _tc_detile(jnp.swapaxes(embeddings, 1, 2))
    eg0 = _sc_gather_paired_half(table3, idx0)
    eg1 = _sc_gather_paired_half(table3, idx1)
    vi = var_idx.reshape(B * V).astype(jnp.int32)

    WT = W.T
    Z = jnp.zeros((2 * F, F), dtype=W.dtype)
    W2T = jnp.concatenate(
        [jnp.concatenate([WT, Z], axis=1), jnp.concatenate([Z, WT], axis=1)],
        axis=0,
    )
    b2 = jnp.tile(b.reshape(2 * F, 1), (1, 2 * F))

    xT = jnp.swapaxes(x, 3, 4)
    outT0 = _tc_film_half(vi, xT, eg0, W2T, b2, half=0)
    outT = _tc_film_half(vi, xT, eg1, W2T, b2, half=1, prev=outT0)
    return jnp.swapaxes(outT, 3, 4)


# speedup vs baseline: 1.1725x; 1.1725x over previous
import functools

import jax
import jax.numpy as jnp
from jax import lax
from jax.experimental import pallas as pl
from jax.experimental.pallas import tpu as pltpu
from jax.experimental.pallas import tpu_sc as plsc

N_NODES = 49152
F = 64
NVARS = 4
B = 2
V = 4

_NC = 2
_NS = 16
_NW = _NC * _NS

_NH = N_NODES // 2
_RH = NVARS * _NH
_NPW = _RH // _NW
_IDX_PER_XFER = 128
_XFERS_PER_BLK = 8
_BLK_ROWS = _IDX_PER_XFER * _XFERS_PER_BLK
_NBLK = _NPW // _BLK_ROWS
_XPW = _NPW // _IDX_PER_XFER


def _sc_gather_paired_half(table, idx3):
    mesh = plsc.VectorSubcoreMesh(core_axis_name="c", subcore_axis_name="s")

    @functools.partial(
        pl.kernel,
        out_type=jax.ShapeDtypeStruct((NVARS // 2, _NH, 2 * F), jnp.float32),
        mesh=mesh,
        compiler_params=pltpu.CompilerParams(use_tc_tiling_on_sc=False),
        scratch_types=[
            pltpu.VMEM((_XPW, _IDX_PER_XFER), jnp.int32),
            pltpu.VMEM((_BLK_ROWS, F), jnp.float32),
            pltpu.SemaphoreType.DMA,
        ],
    )
    def gather_kernel(table_hbm, idx_hbm, out_hbm, idx_v, rows_v, sem):
        wid = lax.axis_index("s") * _NC + lax.axis_index("c")
        u = wid // 8
        node0 = (wid % 8) * _NPW
        p = u // 2
        q = u % 2
        pltpu.sync_copy(idx_hbm.at[wid], idx_v)

        def blk_body(blk, _):
            copies = []
            for j in range(_XFERS_PER_BLK):
                copies.append(pltpu.async_copy(
                    table_hbm.at[idx_v.at[blk * _XFERS_PER_BLK + j]],
                    rows_v.at[pl.ds(j * _IDX_PER_XFER, _IDX_PER_XFER)],
                    sem,
                ))
            for c in copies:
                c.wait()
            pltpu.sync_copy(
                rows_v,
                out_hbm.at[p, pl.ds(node0 + blk * _BLK_ROWS, _BLK_ROWS),
                           pl.ds(q * F, F)],
            )
            return ()

        lax.fori_loop(0, _NBLK, blk_body, (), unroll=False)

    return gather_kernel(table, idx3)


_BN = 8192
_NBH = _NH // _BN


def _film_body(vi_ref, x_ref, eg_ref, w_ref, b_ref, o_ref):
    bv = pl.program_id(1)
    vi = vi_ref[bv]
    pair = eg_ref[vi // 2]
    m_all = lax.dot_general(
        w_ref[...], pair, (((1,), (1,)), ((), ())),
        preferred_element_type=jnp.float32,
    )
    m = jnp.where(vi % 2 == 0, m_all[: 2 * F], m_all[2 * F :])
    m = m + b_ref[:, 0:1]
    o_ref[0, 0, 0] = x_ref[0, 0, 0] * m[:F] + m[F:]


def _film_body_alias(vi_ref, x_ref, eg_ref, w_ref, b_ref, prev_ref, o_ref):
    del prev_ref
    _film_body(vi_ref, x_ref, eg_ref, w_ref, b_ref, o_ref)


def _tc_film_half(vi, xT, eg, W2T, b2, half, prev=None):
    grid = (_NBH, B * V)
    base = half * _NBH

    in_specs = [
        pl.BlockSpec(
            (1, 1, 1, F, _BN),
            lambda n, bv, vi_ref: (bv // V, 0, bv % V, 0, base + n),
        ),
        pl.BlockSpec(
            (NVARS // 2, _BN, 2 * F),
            lambda n, bv, vi_ref: (0, n, 0),
        ),
        pl.BlockSpec((4 * F, 2 * F), lambda n, bv, vi_ref: (0, 0)),
        pl.BlockSpec((2 * F, 2 * F), lambda n, bv, vi_ref: (0, 0)),
    ]
    out_spec = pl.BlockSpec(
        (1, 1, 1, F, _BN),
        lambda n, bv, vi_ref: (bv // V, 0, bv % V, 0, base + n),
    )
    args = [vi, xT, eg, W2T, b2]
    kwargs = {}
    body = _film_body
    if prev is not None:
        in_specs.append(pl.BlockSpec(memory_space=pl.ANY))
        args.append(prev)
        kwargs["input_output_aliases"] = {5: 0}
        body = _film_body_alias
    grid_spec = pltpu.PrefetchScalarGridSpec(
        num_scalar_prefetch=1,
        grid=grid,
        in_specs=in_specs,
        out_specs=out_spec,
    )
    return pl.pallas_call(
        body,
        grid_spec=grid_spec,
        out_shape=jax.ShapeDtypeStruct(xT.shape, xT.dtype),
        **kwargs,
    )(*args)


def kernel(x, var_idx, adjc, embeddings, W, b):
    node_idx = 2 * adjc[:, 0].astype(jnp.int32)
    offs = (jnp.arange(NVARS, dtype=jnp.int32) * (2 * N_NODES))[:, None]
    idx0 = (offs + node_idx[None, :_NH]).reshape(_NW, _XPW, _IDX_PER_XFER)
    idx1 = (offs + node_idx[None, _NH:]).reshape(_NW, _XPW, _IDX_PER_XFER)
    table = lax.pad(embeddings, jnp.float32(0),
                    ((0, 0, 0), (0, 0, 0), (0, F, 0)))
    table = table.reshape(2 * NVARS * N_NODES, F)
    eg0 = _sc_gather_paired_half(table, idx0)
    eg1 = _sc_gather_paired_half(table, idx1)
    vi = var_idx.reshape(B * V).astype(jnp.int32)

    WT = W.T
    Z = jnp.zeros((2 * F, F), dtype=W.dtype)
    W2T = jnp.concatenate(
        [jnp.concatenate([WT, Z], axis=1), jnp.concatenate([Z, WT], axis=1)],
        axis=0,
    )
    b2 = jnp.tile(b.reshape(2 * F, 1), (1, 2 * F))

    xT = jnp.swapaxes(x, 3, 4)
    outT0 = _tc_film_half(vi, xT, eg0, W2T, b2, half=0)
    outT = _tc_film_half(vi, xT, eg1, W2T, b2, half=1, prev=outT0)
    return jnp.swapaxes(outT, 3, 4)


# speedup vs baseline: 1.3251x; 1.1301x over previous
import functools

import jax
import jax.numpy as jnp
from jax import lax
from jax.experimental import pallas as pl
from jax.experimental.pallas import tpu as pltpu
from jax.experimental.pallas import tpu_sc as plsc

N_NODES = 49152
F = 64
NVARS = 4
B = 2
V = 4

_NC = 2
_NS = 16
_NW = _NC * _NS

_NH = N_NODES // 2
_RH = NVARS * _NH
_NPW = _RH // _NW
_IDX_PER_XFER = 128
_XFERS_PER_BLK = 8
_BLK_ROWS = _IDX_PER_XFER * _XFERS_PER_BLK
_NBLK = _NPW // _BLK_ROWS
_XPW = _NPW // _IDX_PER_XFER


def _sc_gather_paired_half(table, idx3):
    mesh = plsc.VectorSubcoreMesh(core_axis_name="c", subcore_axis_name="s")

    @functools.partial(
        pl.kernel,
        out_type=jax.ShapeDtypeStruct((NVARS // 2, _NH, 2 * F), jnp.float32),
        mesh=mesh,
        compiler_params=pltpu.CompilerParams(use_tc_tiling_on_sc=False),
        scratch_types=[
            pltpu.VMEM((_XPW, _IDX_PER_XFER), jnp.int32),
            pltpu.VMEM((_BLK_ROWS, F), jnp.float32),
            pltpu.SemaphoreType.DMA,
        ],
    )
    def gather_kernel(table_hbm, idx_hbm, out_hbm, idx_v, rows_v, sem):
        wid = lax.axis_index("s") * _NC + lax.axis_index("c")
        u = wid // 8
        node0 = (wid % 8) * _NPW
        p = u // 2
        q = u % 2
        pltpu.sync_copy(idx_hbm.at[wid], idx_v)

        def blk_body(blk, _):
            copies = []
            for j in range(_XFERS_PER_BLK):
                copies.append(pltpu.async_copy(
                    table_hbm.at[idx_v.at[blk * _XFERS_PER_BLK + j]],
                    rows_v.at[pl.ds(j * _IDX_PER_XFER, _IDX_PER_XFER)],
                    sem,
                ))
            for c in copies:
                c.wait()
            pltpu.sync_copy(
                rows_v,
                out_hbm.at[p, pl.ds(node0 + blk * _BLK_ROWS, _BLK_ROWS),
                           pl.ds(q * F, F)],
            )
            return ()

        lax.fori_loop(0, _NBLK, blk_body, (), unroll=False)

    return gather_kernel(table, idx3)


_BNT = 4096


def _tpad_body(e_ref, o_ref):
    o_ref[0, :, :F] = jnp.swapaxes(e_ref[0], 0, 1)
    o_ref[0, :, F:] = jnp.zeros((_BNT, F), jnp.float32)


def _tc_tpad(embT):
    return pl.pallas_call(
        _tpad_body,
        grid=(NVARS, N_NODES // _BNT),
        in_specs=[pl.BlockSpec((1, F, _BNT), lambda u, n: (u, 0, n))],
        out_specs=pl.BlockSpec((1, _BNT, 2 * F), lambda u, n: (u, n, 0)),
        out_shape=jax.ShapeDtypeStruct((NVARS, N_NODES, 2 * F), jnp.float32),
    )(embT)


_BN = 8192
_NBH = _NH // _BN


def _film_body(vi_ref, x_ref, eg_ref, w_ref, b_ref, o_ref):
    bv = pl.program_id(1)
    vi = vi_ref[bv]
    pair = eg_ref[vi // 2]
    m_all = lax.dot_general(
        w_ref[...], pair, (((1,), (1,)), ((), ())),
        preferred_element_type=jnp.float32,
    )
    m = jnp.where(vi % 2 == 0, m_all[: 2 * F], m_all[2 * F :])
    m = m + b_ref[:, 0:1]
    o_ref[0, 0, 0] = x_ref[0, 0, 0] * m[:F] + m[F:]


def _film_body_alias(vi_ref, x_ref, eg_ref, w_ref, b_ref, prev_ref, o_ref):
    del prev_ref
    _film_body(vi_ref, x_ref, eg_ref, w_ref, b_ref, o_ref)


def _tc_film_half(vi, xT, eg, W2T, b2, half, prev=None):
    grid = (_NBH, B * V)
    base = half * _NBH

    in_specs = [
        pl.BlockSpec(
            (1, 1, 1, F, _BN),
            lambda n, bv, vi_ref: (bv // V, 0, bv % V, 0, base + n),
        ),
        pl.BlockSpec(
            (NVARS // 2, _BN, 2 * F),
            lambda n, bv, vi_ref: (0, n, 0),
        ),
        pl.BlockSpec((4 * F, 2 * F), lambda n, bv, vi_ref: (0, 0)),
        pl.BlockSpec((2 * F, 2 * F), lambda n, bv, vi_ref: (0, 0)),
    ]
    out_spec = pl.BlockSpec(
        (1, 1, 1, F, _BN),
        lambda n, bv, vi_ref: (bv // V, 0, bv % V, 0, base + n),
    )
    args = [vi, xT, eg, W2T, b2]
    kwargs = {}
    body = _film_body
    if prev is not None:
        in_specs.append(pl.BlockSpec(memory_space=pl.ANY))
        args.append(prev)
        kwargs["input_output_aliases"] = {5: 0}
        body = _film_body_alias
    grid_spec = pltpu.PrefetchScalarGridSpec(
        num_scalar_prefetch=1,
        grid=grid,
        in_specs=in_specs,
        out_specs=out_spec,
    )
    return pl.pallas_call(
        body,
        grid_spec=grid_spec,
        out_shape=jax.ShapeDtypeStruct(xT.shape, xT.dtype),
        **kwargs,
    )(*args)


def kernel(x, var_idx, adjc, embeddings, W, b):
    node_idx = 2 * adjc[:, 0].astype(jnp.int32)
    offs = (jnp.arange(NVARS, dtype=jnp.int32) * (2 * N_NODES))[:, None]
    idx0 = (offs + node_idx[None, :_NH]).reshape(_NW, _XPW, _IDX_PER_XFER)
    idx1 = (offs + node_idx[None, _NH:]).reshape(_NW, _XPW, _IDX_PER_XFER)
    table = _tc_tpad(jnp.swapaxes(embeddings, 1, 2))
    table = table.reshape(2 * NVARS * N_NODES, F)
    eg0 = _sc_gather_paired_half(table, idx0)
    eg1 = _sc_gather_paired_half(table, idx1)
    vi = var_idx.reshape(B * V).astype(jnp.int32)

    WT = W.T
    Z = jnp.zeros((2 * F, F), dtype=W.dtype)
    W2T = jnp.concatenate(
        [jnp.concatenate([WT, Z], axis=1), jnp.concatenate([Z, WT], axis=1)],
        axis=0,
    )
    b2 = jnp.tile(b.reshape(2 * F, 1), (1, 2 * F))

    xT = jnp.swapaxes(x, 3, 4)
    outT0 = _tc_film_half(vi, xT, eg0, W2T, b2, half=0)
    outT = _tc_film_half(vi, xT, eg1, W2T, b2, half=1, prev=outT0)
    return jnp.swapaxes(outT, 3, 4)
